# baseline (speedup 1.0000x reference)
import jax
import jax.numpy as jnp
from jax import lax
from jax.experimental import pallas as pl
from jax.experimental.pallas import tpu as pltpu
from jax.experimental.pallas import tpu_sc as plsc

_B = 16384
_D = 64
_NC = 2
_NS = 16
_NW = _NC * _NS
_RPW = _B // _NW
_CHUNK = 128
_NCHUNK = _RPW // _CHUNK
_L = 16
_CPAD = 17


def _body(user_hbm, item_hbm, theta_hbm, a_hbm, b_hbm, out_hbm,
          idx_u, idx_i, th_v, a_v, b_v, c_v, out_v, sem):
    wid = lax.axis_index("s") * _NC + lax.axis_index("c")
    base = wid * _RPW

    for j in range(_NCHUNK):
        pltpu.sync_copy(user_hbm.at[pl.ds(base + j * _CHUNK, _CHUNK)], idx_u.at[j])
        pltpu.sync_copy(item_hbm.at[pl.ds(base + j * _CHUNK, _CHUNK)], idx_i.at[j])

    cps = []
    for j in range(_NCHUNK):
        sl = pl.ds(j * _CHUNK, _CHUNK)
        cps.append(pltpu.async_copy(theta_hbm.at[idx_u.at[j]], th_v.at[sl], sem))
        cps.append(pltpu.async_copy(a_hbm.at[idx_i.at[j]], a_v.at[sl], sem))
        cps.append(pltpu.async_copy(b_hbm.at[idx_i.at[j]], b_v.at[sl], sem))
    for c in cps:
        c.wait()

    @pl.loop(0, _RPW)
    def _row(i):
        acc = jnp.zeros((_L,), jnp.float32)
        for k in range(_D // _L):
            th = th_v[i, pl.ds(k * _L, _L)]
            ar = a_v[i, pl.ds(k * _L, _L)]
            acc = acc + th / (1.0 + jnp.exp(-ar))
        c_v[pl.ds(i * _CPAD, _L)] = acc

    lane = lax.iota(jnp.int32, _L)
    @pl.loop(0, _RPW // _L)
    def _grp(g):
        rowbase = g * (_L * _CPAD)
        dot = jnp.zeros((_L,), jnp.float32)
        for d in range(_L):
            dot = dot + plsc.load_gather(c_v, [rowbase + lane * _CPAD + d])
        bv = b_v[pl.ds(g * _L, _L)]
        out_v[pl.ds(g * _L, _L)] = 1.0 / (1.0 + jnp.exp(bv - dot))

    pltpu.sync_copy(out_v, out_hbm.at[pl.ds(base, _RPW)])


def kernel(user, item, theta_table, a_table, b_table):
    user = user.astype(jnp.int32)
    item = item.astype(jnp.int32)
    b_flat = b_table.reshape(-1)
    mesh = plsc.VectorSubcoreMesh(
        core_axis_name="c", subcore_axis_name="s",
        num_cores=_NC, num_subcores=_NS)
    ker = pl.kernel(
        _body,
        out_type=jax.ShapeDtypeStruct((_B,), jnp.float32),
        mesh=mesh,
        compiler_params=pltpu.CompilerParams(
            needs_layout_passes=False, use_tc_tiling_on_sc=False),
        scratch_types=[
            pltpu.VMEM((_NCHUNK, _CHUNK), jnp.int32),
            pltpu.VMEM((_NCHUNK, _CHUNK), jnp.int32),
            pltpu.VMEM((_RPW, _D), jnp.float32),
            pltpu.VMEM((_RPW, _D), jnp.float32),
            pltpu.VMEM((_RPW,), jnp.float32),
            pltpu.VMEM((_RPW * _CPAD,), jnp.float32),
            pltpu.VMEM((_RPW,), jnp.float32),
            pltpu.SemaphoreType.DMA,
        ],
    )
    return ker(user, item, theta_table, a_table, b_flat)


# speedup vs baseline: 1.5671x; 1.5671x over previous
import jax
import jax.numpy as jnp
from jax import lax
from jax.experimental import pallas as pl
from jax.experimental.pallas import tpu as pltpu
from jax.experimental.pallas import tpu_sc as plsc

_B = 16384
_D = 64
_NC = 2
_NS = 16
_NW = _NC * _NS
_RPW = _B // _NW
_L = 16
_HALF = _RPW // 2
_CPAD = 17


def _body(user_hbm, item_hbm, theta_hbm, a_hbm, b_hbm, out_hbm,
          idx_u, idx_i, th_v, a_v, b_v, c_v, out_v, sem, semb):
    wid = lax.axis_index("s") * _NC + lax.axis_index("c")
    base = wid * _RPW

    pltpu.sync_copy(user_hbm.at[pl.ds(base, _RPW)], idx_u)
    pltpu.sync_copy(item_hbm.at[pl.ds(base, _RPW)], idx_i)

    bcps = [pltpu.async_copy(b_hbm.at[idx_i.at[pl.ds(j * 128, 128)]],
                             b_v.at[pl.ds(j * 128, 128)], semb)
            for j in range(_RPW // 128)]

    for half in range(2):
        hbase = half * _HALF

        @pl.loop(0, _HALF // _L)
        def _chunk(c):
            iu = idx_u[pl.ds(hbase + c * _L, _L)]
            ii = idx_i[pl.ds(hbase + c * _L, _L)]
            cps = []
            for j in range(_L):
                r = c * _L + j
                cps.append(pltpu.async_copy(theta_hbm.at[iu[j]], th_v.at[r], sem))
                cps.append(pltpu.async_copy(a_hbm.at[ii[j]], a_v.at[r], sem))
            for cp in cps:
                cp.wait()

        @pl.loop(0, _HALF)
        def _row(i):
            acc = jnp.zeros((_L,), jnp.float32)
            for k in range(_D // _L):
                th = th_v[i, pl.ds(k * _L, _L)]
                ar = a_v[i, pl.ds(k * _L, _L)]
                acc = acc + th / (1.0 + jnp.exp(-ar))
            c_v[pl.ds(i * _CPAD, _L)] = acc

        if half == 0:
            for cp in bcps:
                cp.wait()

        lane = lax.iota(jnp.int32, _L)
        @pl.loop(0, _HALF // _L)
        def _grp(g):
            rowbase = g * (_L * _CPAD)
            dot = jnp.zeros((_L,), jnp.float32)
            for d in range(_L):
                dot = dot + plsc.load_gather(c_v, [rowbase + lane * _CPAD + d])
            bv = b_v[pl.ds(hbase + g * _L, _L)]
            out_v[pl.ds(hbase + g * _L, _L)] = 1.0 / (1.0 + jnp.exp(bv - dot))

    pltpu.sync_copy(out_v, out_hbm.at[pl.ds(base, _RPW)])


def kernel(user, item, theta_table, a_table, b_table):
    user = user.astype(jnp.int32)
    item = item.astype(jnp.int32)
    b_lin = b_table.reshape(-1)
    mesh = plsc.VectorSubcoreMesh(
        core_axis_name="c", subcore_axis_name="s",
        num_cores=_NC, num_subcores=_NS)
    ker = pl.kernel(
        _body,
        out_type=jax.ShapeDtypeStruct((_B,), jnp.float32),
        mesh=mesh,
        compiler_params=pltpu.CompilerParams(needs_layout_passes=False),
        scratch_types=[
            pltpu.VMEM((_RPW,), jnp.int32),
            pltpu.VMEM((_RPW,), jnp.int32),
            pltpu.VMEM((_HALF, _D), jnp.float32),
            pltpu.VMEM((_HALF, _D), jnp.float32),
            pltpu.VMEM((_RPW,), jnp.float32),
            pltpu.VMEM((_HALF * _CPAD,), jnp.float32),
            pltpu.VMEM((_RPW,), jnp.float32),
            pltpu.SemaphoreType.DMA,
            pltpu.SemaphoreType.DMA,
        ],
    )
    return ker(user, item, theta_table, a_table, b_lin)
